# baseline (speedup 1.0000x reference)
import functools

import jax
import jax.numpy as jnp
from jax.experimental import pallas as pl
from jax.experimental.pallas import tpu as pltpu

LANE = 128
SUBLANE = 8
TC = 32
NUM_LABELS = 17
OUT_DTYPE = jnp.bfloat16


def _round_up(x, m):
    return ((x + m - 1) // m) * m


def _vmem_limit_bytes():
    return 48 * 1024 * 1024


def _layer_kernel(*refs, hidden, batch, tc, nc, kin, mask_y, proj_out):
    xf_refs = refs[:kin]
    xb_refs = refs[kin:2 * kin]
    wih_ref, b_ref, whh_ref, len_ref = refs[2 * kin:2 * kin + 4]
    pos = 2 * kin + 4
    if proj_out:
        wout_ref = refs[pos]
        pos += 1
    yf_ref, yb_ref = refs[pos:pos + 2]
    gxf_sc, gxb_sc, hf_sc, cf_sc, hb_sc, cb_sc = refs[pos + 2:]

    s = pl.program_id(0)

    @pl.when(s == 0)
    def _init():
        hf_sc[...] = jnp.zeros_like(hf_sc)
        cf_sc[...] = jnp.zeros_like(cf_sc)
        hb_sc[...] = jnp.zeros_like(hb_sc)
        cb_sc[...] = jnp.zeros_like(cb_sc)

    @pl.when(s < nc)
    def _proj():
        af = jnp.dot(xf_refs[0][...], wih_ref[0, 0],
                     preferred_element_type=jnp.float32)
        ab = jnp.dot(xb_refs[0][...], wih_ref[1, 0],
                     preferred_element_type=jnp.float32)
        for k in range(1, kin):
            af = af + jnp.dot(xf_refs[k][...], wih_ref[0, k],
                              preferred_element_type=jnp.float32)
            ab = ab + jnp.dot(xb_refs[k][...], wih_ref[1, k],
                              preferred_element_type=jnp.float32)
        gxf_sc[s % 2] = af + b_ref[0]
        gxb_sc[s % 2] = ab + b_ref[1]

    lens = len_ref[...]
    whh_f = whh_ref[0]
    whh_b = whh_ref[1]

    @pl.when(s > 0)
    def _recurrence():
        buf = (s - 1) % 2
        base_f = (s - 1) * tc
        base_b = (nc - s) * tc

        def cell(gx_sc, whh, r, t_glob, h, c, mask_carry, y_ref, dd):
            r0 = r * batch
            gates = gx_sc[buf, r0:r0 + batch, :] + jnp.dot(
                h, whh, preferred_element_type=jnp.float32)
            i_g = jax.nn.sigmoid(gates[:, 0 * hidden:1 * hidden])
            f_g = jax.nn.sigmoid(gates[:, 1 * hidden:2 * hidden])
            g_g = jnp.tanh(gates[:, 2 * hidden:3 * hidden])
            o_g = jax.nn.sigmoid(gates[:, 3 * hidden:4 * hidden])
            c_new = f_g * c + i_g * g_g
            h_new = (o_g * jnp.tanh(c_new)).astype(jnp.bfloat16)
            if mask_y or mask_carry:
                valid = lens > t_glob
            if mask_y:
                y = jnp.where(valid, h_new, jnp.zeros_like(h_new))
            else:
                y = h_new
            if proj_out:
                y_ref[r0:r0 + batch, :] = jnp.dot(
                    y, wout_ref[dd], preferred_element_type=jnp.float32)
            else:
                y_ref[r0:r0 + batch, :] = y
            if mask_carry:
                return (jnp.where(valid, h_new, h),
                        jnp.where(valid, c_new, c))
            return h_new, c_new

        hf, cf = hf_sc[...], cf_sc[...]
        hb, cb = hb_sc[...], cb_sc[...]
        for k in range(tc):
            hf, cf = cell(gxf_sc, whh_f, k, base_f + k, hf, cf,
                          False, yf_ref, 0)
            hb, cb = cell(gxb_sc, whh_b, tc - 1 - k, base_b + tc - 1 - k,
                          hb, cb, True, yb_ref, 1)
        hf_sc[...] = hf
        cf_sc[...] = cf
        hb_sc[...] = hb
        cb_sc[...] = cb


def _bilstm_layer(xs, wih, b, whh, lens2d, wout, *, seq_len, batch, tc,
                  mask_y):
    kin = len(xs)
    m_rows, din = xs[0].shape
    g = wih.shape[-1]
    hp = g // 4
    nc = seq_len // tc
    rows = tc * batch
    proj_out = wout is not None
    out_w = wout.shape[-1] if proj_out else hp
    out_dt = jnp.float32 if proj_out else OUT_DTYPE

    def fmap(s):
        return (jnp.minimum(s, nc - 1), 0)

    def bmap(s):
        return (nc - 1 - jnp.minimum(s, nc - 1), 0)

    def yfmap(s):
        return (jnp.maximum(s - 1, 0), 0)

    def ybmap(s):
        return (nc - 1 - jnp.maximum(s - 1, 0), 0)

    in_specs = ([pl.BlockSpec((rows, din), fmap) for _ in xs]
                + [pl.BlockSpec((rows, din), bmap) for _ in xs]
                + [
        pl.BlockSpec((2, kin, din, g), lambda s: (0, 0, 0, 0)),
        pl.BlockSpec((2, 1, g), lambda s: (0, 0, 0)),
        pl.BlockSpec((2, hp, g), lambda s: (0, 0, 0)),
        pl.BlockSpec((batch, 1), lambda s: (0, 0)),
    ])
    if proj_out:
        in_specs.append(pl.BlockSpec((2, hp, out_w), lambda s: (0, 0, 0)))

    grid_spec = pltpu.PrefetchScalarGridSpec(
        num_scalar_prefetch=0,
        grid=(nc + 1,),
        in_specs=in_specs,
        out_specs=[
            pl.BlockSpec((rows, out_w), yfmap),
            pl.BlockSpec((rows, out_w), ybmap),
        ],
        scratch_shapes=[
            pltpu.VMEM((2, rows, g), jnp.float32),
            pltpu.VMEM((2, rows, g), jnp.float32),
            pltpu.VMEM((batch, hp), jnp.bfloat16),
            pltpu.VMEM((batch, hp), jnp.float32),
            pltpu.VMEM((batch, hp), jnp.bfloat16),
            pltpu.VMEM((batch, hp), jnp.float32),
        ],
    )
    operands = list(xs) + list(xs) + [wih, b, whh, lens2d]
    if proj_out:
        operands.append(wout)
    return pl.pallas_call(
        functools.partial(_layer_kernel, hidden=hp, batch=batch, tc=tc,
                          nc=nc, kin=kin, mask_y=mask_y, proj_out=proj_out),
        out_shape=[jax.ShapeDtypeStruct((m_rows, out_w), out_dt),
                   jax.ShapeDtypeStruct((m_rows, out_w), out_dt)],
        grid_spec=grid_spec,
        compiler_params=pltpu.CompilerParams(
            dimension_semantics=("arbitrary",),
            vmem_limit_bytes=_vmem_limit_bytes()),
    )(*operands)


def kernel(token_ids, lengths, embedding, lstm0_w_ih, lstm0_w_hh, lstm0_b,
           lstm1_w_ih, lstm1_w_hh, lstm1_b, w_out, b_out):
    B, T = token_ids.shape
    tc = TC
    Bp = _round_up(max(B, SUBLANE), SUBLANE)
    Tp = _round_up(T, tc)
    M = Tp * Bp

    ids = token_ids
    lens = lengths.astype(jnp.int32)
    if (Bp, Tp) != (B, T):
        ids = jnp.zeros((Bp, Tp), token_ids.dtype).at[:B, :T].set(token_ids)
        lens = jnp.zeros((Bp,), jnp.int32).at[:B].set(lens)
    lens2d = lens.reshape(Bp, 1)

    emb = jnp.take(embedding, ids.T, axis=0)
    x = emb.reshape(M, emb.shape[-1])

    yf, yb = _bilstm_layer([x], lstm0_w_ih, lstm0_b, lstm0_w_hh, lens2d,
                           None, seq_len=Tp, batch=Bp, tc=tc, mask_y=False)
    pf, pb = _bilstm_layer([yf, yb], lstm1_w_ih, lstm1_b, lstm1_w_hh, lens2d,
                           w_out, seq_len=Tp, batch=Bp, tc=tc, mask_y=True)

    logits_p = pf + pb + b_out
    logits = logits_p.reshape(Tp, Bp, -1)[:T, :B, :NUM_LABELS]
    return jnp.transpose(logits, (1, 0, 2))


# speedup vs baseline: 1.2211x; 1.2211x over previous
import functools

import jax
import jax.numpy as jnp
from jax.experimental import pallas as pl
from jax.experimental.pallas import tpu as pltpu

LANE = 128
SUBLANE = 8
TC = 32
NUM_LABELS = 17
OUT_DTYPE = jnp.bfloat16


def _round_up(x, m):
    return ((x + m - 1) // m) * m


def _vmem_limit_bytes():
    return 48 * 1024 * 1024


def _layer_kernel(*refs, hidden, batch, tc, nc, kin, mask_y, proj_out):
    xf_refs = refs[:kin]
    xb_refs = refs[kin:2 * kin]
    wih_ref, b_ref, whh_ref, len_ref = refs[2 * kin:2 * kin + 4]
    pos = 2 * kin + 4
    if proj_out:
        wout_ref = refs[pos]
        pos += 1
    yf_ref, yb_ref = refs[pos:pos + 2]
    gxf_sc, gxb_sc, hf_sc, cf_sc, hb_sc, cb_sc = refs[pos + 2:]

    s = pl.program_id(0)

    @pl.when(s == 0)
    def _init():
        hf_sc[...] = jnp.zeros_like(hf_sc)
        cf_sc[...] = jnp.zeros_like(cf_sc)
        hb_sc[...] = jnp.zeros_like(hb_sc)
        cb_sc[...] = jnp.zeros_like(cb_sc)

    @pl.when(s < nc)
    def _proj():
        af = jnp.dot(xf_refs[0][...], wih_ref[0, 0],
                     preferred_element_type=jnp.float32)
        ab = jnp.dot(xb_refs[0][...], wih_ref[1, 0],
                     preferred_element_type=jnp.float32)
        for k in range(1, kin):
            af = af + jnp.dot(xf_refs[k][...], wih_ref[0, k],
                              preferred_element_type=jnp.float32)
            ab = ab + jnp.dot(xb_refs[k][...], wih_ref[1, k],
                              preferred_element_type=jnp.float32)
        gxf_sc[s % 2] = af + b_ref[0]
        gxb_sc[s % 2] = ab + b_ref[1]

    lens = len_ref[...]
    whh_f = whh_ref[0]
    whh_b = whh_ref[1]

    @pl.when(s > 0)
    def _recurrence():
        buf = (s - 1) % 2
        base_f = (s - 1) * tc
        base_b = (nc - s) * tc

        def cell(gx_sc, whh, r, t_glob, h, c, mask_carry, y_ref, dd):
            r0 = r * batch
            gates = gx_sc[buf, r0:r0 + batch, :] + jnp.dot(
                h, whh, preferred_element_type=jnp.float32)
            i_g = jax.nn.sigmoid(gates[:, 0 * hidden:1 * hidden])
            f_g = jax.nn.sigmoid(gates[:, 1 * hidden:2 * hidden])
            g_g = jnp.tanh(gates[:, 2 * hidden:3 * hidden])
            o_g = jax.nn.sigmoid(gates[:, 3 * hidden:4 * hidden])
            c_new = f_g * c + i_g * g_g
            h_new = (o_g * jnp.tanh(c_new)).astype(jnp.bfloat16)
            if mask_y or mask_carry:
                valid = lens > t_glob
            if mask_y:
                y = jnp.where(valid, h_new, jnp.zeros_like(h_new))
            else:
                y = h_new
            if proj_out:
                y_ref[r0:r0 + batch, :] = jnp.dot(
                    y, wout_ref[dd], preferred_element_type=jnp.float32)
            else:
                y_ref[r0:r0 + batch, :] = y
            if mask_carry:
                return (jnp.where(valid, h_new, h),
                        jnp.where(valid, c_new, c))
            return h_new, c_new

        hf, cf = hf_sc[...], cf_sc[...]
        hb, cb = hb_sc[...], cb_sc[...]
        for k in range(tc):
            hf, cf = cell(gxf_sc, whh_f, k, base_f + k, hf, cf,
                          False, yf_ref, 0)
            hb, cb = cell(gxb_sc, whh_b, tc - 1 - k, base_b + tc - 1 - k,
                          hb, cb, True, yb_ref, 1)
        hf_sc[...] = hf
        cf_sc[...] = cf
        hb_sc[...] = hb
        cb_sc[...] = cb


def _bilstm_layer(xs, wih, b, whh, lens2d, wout, *, seq_len, batch, tc,
                  mask_y):
    kin = len(xs)
    m_rows, din = xs[0].shape
    g = wih.shape[-1]
    hp = g // 4
    nc = seq_len // tc
    rows = tc * batch
    proj_out = wout is not None
    out_w = wout.shape[-1] if proj_out else hp
    out_dt = jnp.float32 if proj_out else OUT_DTYPE

    def fmap(s):
        return (jnp.minimum(s, nc - 1), 0)

    def bmap(s):
        return (nc - 1 - jnp.minimum(s, nc - 1), 0)

    def yfmap(s):
        return (jnp.maximum(s - 1, 0), 0)

    def ybmap(s):
        return (nc - 1 - jnp.maximum(s - 1, 0), 0)

    in_specs = ([pl.BlockSpec((rows, din), fmap) for _ in xs]
                + [pl.BlockSpec((rows, din), bmap) for _ in xs]
                + [
        pl.BlockSpec((2, kin, din, g), lambda s: (0, 0, 0, 0)),
        pl.BlockSpec((2, 1, g), lambda s: (0, 0, 0)),
        pl.BlockSpec((2, hp, g), lambda s: (0, 0, 0)),
        pl.BlockSpec((batch, 1), lambda s: (0, 0)),
    ])
    if proj_out:
        in_specs.append(pl.BlockSpec((2, hp, out_w), lambda s: (0, 0, 0)))

    grid_spec = pltpu.PrefetchScalarGridSpec(
        num_scalar_prefetch=0,
        grid=(nc + 1,),
        in_specs=in_specs,
        out_specs=[
            pl.BlockSpec((rows, out_w), yfmap),
            pl.BlockSpec((rows, out_w), ybmap),
        ],
        scratch_shapes=[
            pltpu.VMEM((2, rows, g), jnp.float32),
            pltpu.VMEM((2, rows, g), jnp.float32),
            pltpu.VMEM((batch, hp), jnp.bfloat16),
            pltpu.VMEM((batch, hp), jnp.float32),
            pltpu.VMEM((batch, hp), jnp.bfloat16),
            pltpu.VMEM((batch, hp), jnp.float32),
        ],
    )
    operands = list(xs) + list(xs) + [wih, b, whh, lens2d]
    if proj_out:
        operands.append(wout)
    return pl.pallas_call(
        functools.partial(_layer_kernel, hidden=hp, batch=batch, tc=tc,
                          nc=nc, kin=kin, mask_y=mask_y, proj_out=proj_out),
        out_shape=[jax.ShapeDtypeStruct((m_rows, out_w), out_dt),
                   jax.ShapeDtypeStruct((m_rows, out_w), out_dt)],
        grid_spec=grid_spec,
        compiler_params=pltpu.CompilerParams(
            dimension_semantics=("arbitrary",),
            vmem_limit_bytes=_vmem_limit_bytes()),
    )(*operands)


def _proj_kernel(xf_ref, xb_ref, w_ref, b_ref, o_ref):
    acc = jnp.dot(xf_ref[...], w_ref[0], preferred_element_type=jnp.float32)
    acc = acc + jnp.dot(xb_ref[...], w_ref[1],
                        preferred_element_type=jnp.float32)
    o_ref[...] = acc + b_ref[...]


def _output_projection(xf, xb, w, b, tm=1024):
    m_rows, hp = xf.shape
    lp = w.shape[-1]
    while m_rows % tm:
        tm //= 2
    grid_spec = pltpu.PrefetchScalarGridSpec(
        num_scalar_prefetch=0,
        grid=(m_rows // tm,),
        in_specs=[
            pl.BlockSpec((tm, hp), lambda m: (m, 0)),
            pl.BlockSpec((tm, hp), lambda m: (m, 0)),
            pl.BlockSpec((2, hp, lp), lambda m: (0, 0, 0)),
            pl.BlockSpec((1, lp), lambda m: (0, 0)),
        ],
        out_specs=pl.BlockSpec((tm, lp), lambda m: (m, 0)),
    )
    return pl.pallas_call(
        _proj_kernel,
        out_shape=jax.ShapeDtypeStruct((m_rows, lp), jnp.float32),
        grid_spec=grid_spec,
        compiler_params=pltpu.CompilerParams(
            dimension_semantics=("arbitrary",),
            vmem_limit_bytes=_vmem_limit_bytes()),
    )(xf, xb, w, b)


def kernel(token_ids, lengths, embedding, lstm0_w_ih, lstm0_w_hh, lstm0_b,
           lstm1_w_ih, lstm1_w_hh, lstm1_b, w_out, b_out):
    B, T = token_ids.shape
    tc = TC
    Bp = _round_up(max(B, SUBLANE), SUBLANE)
    Tp = _round_up(T, tc)
    M = Tp * Bp

    ids = token_ids
    lens = lengths.astype(jnp.int32)
    if (Bp, Tp) != (B, T):
        ids = jnp.zeros((Bp, Tp), token_ids.dtype).at[:B, :T].set(token_ids)
        lens = jnp.zeros((Bp,), jnp.int32).at[:B].set(lens)
    lens2d = lens.reshape(Bp, 1)

    emb = jnp.take(embedding, ids.T, axis=0)
    x = emb.reshape(M, emb.shape[-1])

    yf, yb = _bilstm_layer([x], lstm0_w_ih, lstm0_b, lstm0_w_hh, lens2d,
                           None, seq_len=Tp, batch=Bp, tc=tc, mask_y=False)
    yf, yb = _bilstm_layer([yf, yb], lstm1_w_ih, lstm1_b, lstm1_w_hh, lens2d,
                           None, seq_len=Tp, batch=Bp, tc=tc, mask_y=True)

    logits_p = _output_projection(yf, yb, w_out, b_out)
    logits = logits_p.reshape(Tp, Bp, -1)[:T, :B, :NUM_LABELS]
    return jnp.transpose(logits, (1, 0, 2))
